# baseline (speedup 1.0000x reference)
import functools

import jax
import jax.numpy as jnp
from jax import lax
from jax.experimental import pallas as pl
from jax.experimental.pallas import tpu as pltpu
from jax.experimental.pallas import tpu_sc as plsc

B, N, D, K = 4, 4096, 3, 20
Q = 1024
NB = N // Q
I32MAX = 0x7FFFFFFF


def _knn_body(xp_ref, xt_ref, idx_ref):
    b = pl.program_id(0)
    xp = xp_ref[0]
    xt = xt_ref[0]
    q0, q1, q2 = xp[:, 0:1], xp[:, 1:2], xp[:, 2:3]
    k0, k1, k2 = xt[0:1, :], xt[1:2, :], xt[2:3, :]
    qsq = q0 * q0 + q1 * q1 + q2 * q2
    ksq = k0 * k0 + k1 * k1 + k2 * k2
    inner = q0 * k0 + q1 * k1 + q2 * k2
    dist = (qsq + (-2.0) * inner) + ksq
    bits = lax.bitcast_convert_type(dist, jnp.int32)
    mono = bits ^ (jnp.right_shift(bits, 31) & 0x7FFFFFFF)
    jcol = lax.broadcasted_iota(jnp.int32, (Q, N), 1)
    key = (mono & -4096) | jcol
    ms = []
    for _ in range(K):
        m = jnp.min(key, axis=1, keepdims=True)
        ms.append(m)
        key = jnp.where(key == m, I32MAX, key)
    sel = jnp.concatenate(ms, axis=1)
    idx_ref[0] = (sel & 4095) + b * N


def _knn_call(x, xt):
    return pl.pallas_call(
        _knn_body,
        grid=(B, NB),
        in_specs=[
            pl.BlockSpec((1, Q, D), lambda b, nb: (b, nb, 0)),
            pl.BlockSpec((1, D, N), lambda b, nb: (b, 0, 0)),
        ],
        out_specs=pl.BlockSpec((1, Q, K), lambda b, nb: (b, nb, 0)),
        out_shape=jax.ShapeDtypeStruct((B, N, K), jnp.int32),
    )(x, xt)


_NW = 32
_TOT = B * N * K
_CH = _TOT // _NW
_NF = B * N


def _sc_gather(x0, x1, x2, idx_flat):
    mesh = plsc.VectorSubcoreMesh(core_axis_name="c", subcore_axis_name="s")

    @functools.partial(
        pl.kernel,
        mesh=mesh,
        compiler_params=pltpu.CompilerParams(needs_layout_passes=False),
        out_type=[jax.ShapeDtypeStruct((_TOT,), jnp.float32)] * 3,
        scratch_types=[
            pltpu.VMEM((_NF,), jnp.float32),
            pltpu.VMEM((_NF,), jnp.float32),
            pltpu.VMEM((_NF,), jnp.float32),
            pltpu.VMEM((_CH,), jnp.int32),
            pltpu.VMEM((_CH,), jnp.float32),
            pltpu.VMEM((_CH,), jnp.float32),
            pltpu.VMEM((_CH,), jnp.float32),
        ],
    )
    def gather_k(x0h, x1h, x2h, idxh, o0h, o1h, o2h,
                 x0v, x1v, x2v, idxv, o0v, o1v, o2v):
        wid = lax.axis_index("s") * 2 + lax.axis_index("c")
        base = wid * _CH
        pltpu.sync_copy(x0h, x0v)
        pltpu.sync_copy(x1h, x1v)
        pltpu.sync_copy(x2h, x2v)
        pltpu.sync_copy(idxh.at[pl.ds(base, _CH)], idxv)

        def body(g, _):
            off = pl.multiple_of(g * 16, 16)
            iv = idxv[pl.ds(off, 16)]
            o0v[pl.ds(off, 16)] = plsc.load_gather(x0v, [iv])
            o1v[pl.ds(off, 16)] = plsc.load_gather(x1v, [iv])
            o2v[pl.ds(off, 16)] = plsc.load_gather(x2v, [iv])
            return _

        lax.fori_loop(0, _CH // 16, body, None)
        pltpu.sync_copy(o0v, o0h.at[pl.ds(base, _CH)])
        pltpu.sync_copy(o1v, o1h.at[pl.ds(base, _CH)])
        pltpu.sync_copy(o2v, o2h.at[pl.ds(base, _CH)])

    return gather_k(x0, x1, x2, idx_flat)


def _mlp_body(xp_ref, n0_ref, n1_ref, n2_ref, w1_ref, b1_ref, w2_ref, b2_ref,
              w3_ref, b3_ref, g_ref):
    nb = pl.program_id(1)
    xp = xp_ref[0]
    w1 = w1_ref[...]
    wd = w1[0:3, :] - w1[3:6, :]
    cb = (xp[:, 0:1] * wd[0:1, :] + xp[:, 1:2] * wd[1:2, :]
          + xp[:, 2:3] * wd[2:3, :]) + b1_ref[...]
    n0, n1, n2 = n0_ref[0], n1_ref[0], n2_ref[0]
    w2 = w2_ref[...]
    b2 = b2_ref[...]
    h1s = []
    for k in range(K):
        bn = (n0[:, k:k + 1] * w1[3:4, :] + n1[:, k:k + 1] * w1[4:5, :]
              + n2[:, k:k + 1] * w1[5:6, :])
        h1s.append(jnp.maximum(cb + bn, 0.0))
    h1 = jnp.concatenate(h1s, axis=0)
    h2 = jnp.maximum(
        jnp.dot(h1, w2, preferred_element_type=jnp.float32) + b2, 0.0)
    acc = h2[0:Q]
    for k in range(1, K):
        acc = jnp.maximum(acc, h2[k * Q:(k + 1) * Q])
    h3 = jnp.maximum(
        jnp.dot(acc, w3_ref[...], preferred_element_type=jnp.float32)
        + b3_ref[...], 0.0)
    part = jnp.max(h3, axis=0, keepdims=True)[None]

    @pl.when(nb == 0)
    def _():
        g_ref[...] = part

    @pl.when(nb != 0)
    def _():
        g_ref[...] = jnp.maximum(g_ref[...], part)


def _mlp_call(x, n0, n1, n2, w1, b1, w2, b2, w3, b3):
    full = lambda shape: pl.BlockSpec(shape, lambda b, nb: tuple(0 for _ in shape))
    return pl.pallas_call(
        _mlp_body,
        grid=(B, NB),
        in_specs=[
            pl.BlockSpec((1, Q, D), lambda b, nb: (b, nb, 0)),
            pl.BlockSpec((1, Q, K), lambda b, nb: (b, nb, 0)),
            pl.BlockSpec((1, Q, K), lambda b, nb: (b, nb, 0)),
            pl.BlockSpec((1, Q, K), lambda b, nb: (b, nb, 0)),
            full((6, 64)), full((1, 64)),
            full((64, 128)), full((1, 128)),
            full((128, 1024)), full((1, 1024)),
        ],
        out_specs=pl.BlockSpec((1, 1, 1024), lambda b, nb: (b, 0, 0)),
        out_shape=jax.ShapeDtypeStruct((B, 1, 1024), jnp.float32),
    )(x, n0, n1, n2, w1, b1, w2, b2, w3, b3)


def _tail_body(g_ref, w1_ref, b1_ref, w2_ref, b2_ref, tw_ref, tb_ref, t_ref):
    g = g_ref[...]
    g1 = jnp.maximum(
        jnp.dot(g, w1_ref[...], preferred_element_type=jnp.float32)
        + b1_ref[...], 0.0)
    g2 = jnp.maximum(
        jnp.dot(g1, w2_ref[...], preferred_element_type=jnp.float32)
        + b2_ref[...], 0.0)
    t_ref[...] = jnp.dot(g2, tw_ref[...],
                         preferred_element_type=jnp.float32) + tb_ref[...]


def _tail_call(g, w1, b1, w2, b2, tw, tb):
    full = lambda shape: pl.BlockSpec(shape, lambda: tuple(0 for _ in shape))
    return pl.pallas_call(
        _tail_body,
        grid=(),
        in_specs=[full((B, 1024)), full((1024, 512)), full((1, 512)),
                  full((512, 256)), full((1, 256)),
                  full((256, 9)), full((1, 9))],
        out_specs=full((B, 9)),
        out_shape=jax.ShapeDtypeStruct((B, 9), jnp.float32),
    )(g, w1, b1, w2, b2, tw, tb)


def _apply_body(xp_ref, t_ref, o_ref):
    xp = xp_ref[0]
    t = t_ref[0]
    cols = []
    for e in range(3):
        col = (xp[:, 0:1] * t[0:1, e:e + 1]
               + xp[:, 1:2] * t[0:1, 3 + e:4 + e]
               + xp[:, 2:3] * t[0:1, 6 + e:7 + e])
        cols.append(col)
    o_ref[0] = jnp.concatenate(cols, axis=1)


def _apply_call(x, t):
    return pl.pallas_call(
        _apply_body,
        grid=(B, NB),
        in_specs=[
            pl.BlockSpec((1, Q, D), lambda b, nb: (b, nb, 0)),
            pl.BlockSpec((1, 1, 9), lambda b, nb: (b, 0, 0)),
        ],
        out_specs=pl.BlockSpec((1, Q, D), lambda b, nb: (b, nb, 0)),
        out_shape=jax.ShapeDtypeStruct((B, N, D), jnp.float32),
    )(x, t)


def kernel(inputs, conv1_w, conv1_b, conv2_w, conv2_b, conv3_w, conv3_b,
           dense1_w, dense1_b, dense2_w, dense2_b, transform_w, transform_b):
    x = inputs
    xt = jnp.transpose(x, (0, 2, 1))
    idx = _knn_call(x, xt)
    planes = jnp.transpose(x, (2, 0, 1)).reshape(3, B * N)
    n0, n1, n2 = _sc_gather(planes[0], planes[1], planes[2],
                            idx.reshape(-1))
    n0 = n0.reshape(B, N, K)
    n1 = n1.reshape(B, N, K)
    n2 = n2.reshape(B, N, K)
    g = _mlp_call(x, n0, n1, n2, conv1_w, conv1_b.reshape(1, 64),
                  conv2_w, conv2_b.reshape(1, 128),
                  conv3_w, conv3_b.reshape(1, 1024))
    t = _tail_call(g.reshape(B, 1024), dense1_w, dense1_b.reshape(1, 512),
                   dense2_w, dense2_b.reshape(1, 256),
                   transform_w, transform_b.reshape(1, 9))
    return _apply_call(x, t.reshape(B, 1, 9))


# speedup vs baseline: 1.0703x; 1.0703x over previous
import functools

import jax
import jax.numpy as jnp
from jax import lax
from jax.experimental import pallas as pl
from jax.experimental.pallas import tpu as pltpu
from jax.experimental.pallas import tpu_sc as plsc

B, N, D, K = 4, 4096, 3, 20
Q = 256
NB = N // Q
I32MAX = 0x7FFFFFFF


def _knn_body(xp_ref, xt_ref, idx_ref):
    b = pl.program_id(0)
    xp = xp_ref[0]
    xt = xt_ref[0]
    q0, q1, q2 = xp[:, 0:1], xp[:, 1:2], xp[:, 2:3]
    k0, k1, k2 = xt[0:1, :], xt[1:2, :], xt[2:3, :]
    qsq = q0 * q0 + q1 * q1 + q2 * q2
    ksq = k0 * k0 + k1 * k1 + k2 * k2
    inner = q0 * k0 + q1 * k1 + q2 * k2
    dist = (qsq + (-2.0) * inner) + ksq
    bits = lax.bitcast_convert_type(dist, jnp.int32)
    mono = bits ^ (jnp.right_shift(bits, 31) & 0x7FFFFFFF)
    jcol = lax.broadcasted_iota(jnp.int32, (Q, N), 1)
    key = (mono & -4096) | jcol
    ms = []
    for _ in range(K):
        m = jnp.min(key, axis=1, keepdims=True)
        ms.append(m)
        key = jnp.where(key == m, I32MAX, key)
    sel = jnp.concatenate(ms, axis=1)
    idx_ref[0] = (sel & 4095) + b * N


def _knn_call(x, xt):
    return pl.pallas_call(
        _knn_body,
        grid=(B, NB),
        in_specs=[
            pl.BlockSpec((1, Q, D), lambda b, nb: (b, nb, 0)),
            pl.BlockSpec((1, D, N), lambda b, nb: (b, 0, 0)),
        ],
        out_specs=pl.BlockSpec((1, Q, K), lambda b, nb: (b, nb, 0)),
        out_shape=jax.ShapeDtypeStruct((B, N, K), jnp.int32),
    )(x, xt)


_NW = 32
_TOT = B * N * K
_CH = _TOT // _NW
_NF = B * N


def _sc_gather(x0, x1, x2, idx_flat):
    mesh = plsc.VectorSubcoreMesh(core_axis_name="c", subcore_axis_name="s")

    @functools.partial(
        pl.kernel,
        mesh=mesh,
        compiler_params=pltpu.CompilerParams(needs_layout_passes=False),
        out_type=[jax.ShapeDtypeStruct((_TOT,), jnp.float32)] * 3,
        scratch_types=[
            pltpu.VMEM((_NF,), jnp.float32),
            pltpu.VMEM((_NF,), jnp.float32),
            pltpu.VMEM((_NF,), jnp.float32),
            pltpu.VMEM((_CH,), jnp.int32),
            pltpu.VMEM((_CH,), jnp.float32),
            pltpu.VMEM((_CH,), jnp.float32),
            pltpu.VMEM((_CH,), jnp.float32),
        ],
    )
    def gather_k(x0h, x1h, x2h, idxh, o0h, o1h, o2h,
                 x0v, x1v, x2v, idxv, o0v, o1v, o2v):
        wid = lax.axis_index("s") * 2 + lax.axis_index("c")
        base = wid * _CH
        pltpu.sync_copy(x0h, x0v)
        pltpu.sync_copy(x1h, x1v)
        pltpu.sync_copy(x2h, x2v)
        pltpu.sync_copy(idxh.at[pl.ds(base, _CH)], idxv)

        def body(g, _):
            off = pl.multiple_of(g * 16, 16)
            iv = idxv[pl.ds(off, 16)]
            o0v[pl.ds(off, 16)] = plsc.load_gather(x0v, [iv])
            o1v[pl.ds(off, 16)] = plsc.load_gather(x1v, [iv])
            o2v[pl.ds(off, 16)] = plsc.load_gather(x2v, [iv])
            return _

        lax.fori_loop(0, _CH // 16, body, None)
        pltpu.sync_copy(o0v, o0h.at[pl.ds(base, _CH)])
        pltpu.sync_copy(o1v, o1h.at[pl.ds(base, _CH)])
        pltpu.sync_copy(o2v, o2h.at[pl.ds(base, _CH)])

    return gather_k(x0, x1, x2, idx_flat)


def _mlp_body(xp_ref, n0_ref, n1_ref, n2_ref, w1_ref, b1_ref, w2_ref, b2_ref,
              w3_ref, b3_ref, g_ref):
    nb = pl.program_id(1)
    xp = xp_ref[0]
    w1 = w1_ref[...]
    wd = w1[0:3, :] - w1[3:6, :]
    cb = (xp[:, 0:1] * wd[0:1, :] + xp[:, 1:2] * wd[1:2, :]
          + xp[:, 2:3] * wd[2:3, :]) + b1_ref[...]
    n0, n1, n2 = n0_ref[0], n1_ref[0], n2_ref[0]
    w2 = w2_ref[...]
    b2 = b2_ref[...]
    h1s = []
    for k in range(K):
        bn = (n0[:, k:k + 1] * w1[3:4, :] + n1[:, k:k + 1] * w1[4:5, :]
              + n2[:, k:k + 1] * w1[5:6, :])
        h1s.append(jnp.maximum(cb + bn, 0.0))
    h1 = jnp.concatenate(h1s, axis=0)
    h2 = jnp.maximum(
        jnp.dot(h1, w2, preferred_element_type=jnp.float32) + b2, 0.0)
    acc = h2[0:Q]
    for k in range(1, K):
        acc = jnp.maximum(acc, h2[k * Q:(k + 1) * Q])
    h3 = jnp.maximum(
        jnp.dot(acc, w3_ref[...], preferred_element_type=jnp.float32)
        + b3_ref[...], 0.0)
    part = jnp.max(h3, axis=0, keepdims=True)[None]

    @pl.when(nb == 0)
    def _():
        g_ref[...] = part

    @pl.when(nb != 0)
    def _():
        g_ref[...] = jnp.maximum(g_ref[...], part)


def _mlp_call(x, n0, n1, n2, w1, b1, w2, b2, w3, b3):
    full = lambda shape: pl.BlockSpec(shape, lambda b, nb: tuple(0 for _ in shape))
    return pl.pallas_call(
        _mlp_body,
        grid=(B, NB),
        in_specs=[
            pl.BlockSpec((1, Q, D), lambda b, nb: (b, nb, 0)),
            pl.BlockSpec((1, Q, K), lambda b, nb: (b, nb, 0)),
            pl.BlockSpec((1, Q, K), lambda b, nb: (b, nb, 0)),
            pl.BlockSpec((1, Q, K), lambda b, nb: (b, nb, 0)),
            full((6, 64)), full((1, 64)),
            full((64, 128)), full((1, 128)),
            full((128, 1024)), full((1, 1024)),
        ],
        out_specs=pl.BlockSpec((1, 1, 1024), lambda b, nb: (b, 0, 0)),
        out_shape=jax.ShapeDtypeStruct((B, 1, 1024), jnp.float32),
    )(x, n0, n1, n2, w1, b1, w2, b2, w3, b3)


def _tail_body(g_ref, w1_ref, b1_ref, w2_ref, b2_ref, tw_ref, tb_ref, t_ref):
    g = g_ref[...]
    g1 = jnp.maximum(
        jnp.dot(g, w1_ref[...], preferred_element_type=jnp.float32)
        + b1_ref[...], 0.0)
    g2 = jnp.maximum(
        jnp.dot(g1, w2_ref[...], preferred_element_type=jnp.float32)
        + b2_ref[...], 0.0)
    t_ref[...] = jnp.dot(g2, tw_ref[...],
                         preferred_element_type=jnp.float32) + tb_ref[...]


def _tail_call(g, w1, b1, w2, b2, tw, tb):
    full = lambda shape: pl.BlockSpec(shape, lambda: tuple(0 for _ in shape))
    return pl.pallas_call(
        _tail_body,
        grid=(),
        in_specs=[full((B, 1024)), full((1024, 512)), full((1, 512)),
                  full((512, 256)), full((1, 256)),
                  full((256, 9)), full((1, 9))],
        out_specs=full((B, 9)),
        out_shape=jax.ShapeDtypeStruct((B, 9), jnp.float32),
    )(g, w1, b1, w2, b2, tw, tb)


def _apply_body(xp_ref, t_ref, o_ref):
    xp = xp_ref[0]
    t = t_ref[0]
    cols = []
    for e in range(3):
        col = (xp[:, 0:1] * t[0:1, e:e + 1]
               + xp[:, 1:2] * t[0:1, 3 + e:4 + e]
               + xp[:, 2:3] * t[0:1, 6 + e:7 + e])
        cols.append(col)
    o_ref[0] = jnp.concatenate(cols, axis=1)


def _apply_call(x, t):
    return pl.pallas_call(
        _apply_body,
        grid=(B, NB),
        in_specs=[
            pl.BlockSpec((1, Q, D), lambda b, nb: (b, nb, 0)),
            pl.BlockSpec((1, 1, 9), lambda b, nb: (b, 0, 0)),
        ],
        out_specs=pl.BlockSpec((1, Q, D), lambda b, nb: (b, nb, 0)),
        out_shape=jax.ShapeDtypeStruct((B, N, D), jnp.float32),
    )(x, t)


def kernel(inputs, conv1_w, conv1_b, conv2_w, conv2_b, conv3_w, conv3_b,
           dense1_w, dense1_b, dense2_w, dense2_b, transform_w, transform_b):
    x = inputs
    xt = jnp.transpose(x, (0, 2, 1))
    idx = _knn_call(x, xt)
    planes = jnp.transpose(x, (2, 0, 1)).reshape(3, B * N)
    n0, n1, n2 = _sc_gather(planes[0], planes[1], planes[2],
                            idx.reshape(-1))
    n0 = n0.reshape(B, N, K)
    n1 = n1.reshape(B, N, K)
    n2 = n2.reshape(B, N, K)
    g = _mlp_call(x, n0, n1, n2, conv1_w, conv1_b.reshape(1, 64),
                  conv2_w, conv2_b.reshape(1, 128),
                  conv3_w, conv3_b.reshape(1, 1024))
    t = _tail_call(g.reshape(B, 1024), dense1_w, dense1_b.reshape(1, 512),
                   dense2_w, dense2_b.reshape(1, 256),
                   transform_w, transform_b.reshape(1, 9))
    return _apply_call(x, t.reshape(B, 1, 9))


# speedup vs baseline: 1.4383x; 1.3438x over previous
import functools

import jax
import jax.numpy as jnp
from jax import lax
from jax.experimental import pallas as pl
from jax.experimental.pallas import tpu as pltpu
from jax.experimental.pallas import tpu_sc as plsc

B, N, D, K = 4, 4096, 3, 20
Q = 512
NB = N // Q
I32MAX = 0x7FFFFFFF


def _knn_body(xp_ref, xt_ref, idx_ref):
    b = pl.program_id(0)
    xp = xp_ref[0]
    xt = xt_ref[0]
    q0, q1, q2 = xp[:, 0:1], xp[:, 1:2], xp[:, 2:3]
    k0, k1, k2 = xt[0:1, :], xt[1:2, :], xt[2:3, :]
    qsq = q0 * q0 + q1 * q1 + q2 * q2
    ksq = k0 * k0 + k1 * k1 + k2 * k2
    inner = q0 * k0 + q1 * k1 + q2 * k2
    dist = (qsq + (-2.0) * inner) + ksq
    dist_c = jnp.maximum(dist, 1e-30)
    bits = lax.bitcast_convert_type(dist_c, jnp.int32)
    jcol = lax.broadcasted_iota(jnp.int32, (Q, N), 1)
    key = lax.bitcast_convert_type((bits & -4096) | jcol, jnp.float32)
    ms = []
    for _ in range(K):
        m = jnp.min(key, axis=1, keepdims=True)
        ms.append(m)
        key = jnp.where(key == m, float("inf"), key)
    sel = lax.bitcast_convert_type(jnp.concatenate(ms, axis=1), jnp.int32)
    idx_ref[0] = (sel & 4095) + b * N


def _knn_call(x, xt):
    return pl.pallas_call(
        _knn_body,
        grid=(B, NB),
        in_specs=[
            pl.BlockSpec((1, Q, D), lambda b, nb: (b, nb, 0)),
            pl.BlockSpec((1, D, N), lambda b, nb: (b, 0, 0)),
        ],
        out_specs=pl.BlockSpec((1, Q, K), lambda b, nb: (b, nb, 0)),
        out_shape=jax.ShapeDtypeStruct((B, N, K), jnp.int32),
    )(x, xt)


_NW = 32
_TOT = B * N * K
_CH = _TOT // _NW
_NF = B * N


def _sc_gather(x0, x1, x2, idx_flat):
    mesh = plsc.VectorSubcoreMesh(core_axis_name="c", subcore_axis_name="s")

    @functools.partial(
        pl.kernel,
        mesh=mesh,
        compiler_params=pltpu.CompilerParams(needs_layout_passes=False),
        out_type=[jax.ShapeDtypeStruct((_TOT,), jnp.float32)] * 3,
        scratch_types=[
            pltpu.VMEM((_NF,), jnp.float32),
            pltpu.VMEM((_NF,), jnp.float32),
            pltpu.VMEM((_NF,), jnp.float32),
            pltpu.VMEM((_CH,), jnp.int32),
            pltpu.VMEM((_CH,), jnp.float32),
            pltpu.VMEM((_CH,), jnp.float32),
            pltpu.VMEM((_CH,), jnp.float32),
        ],
    )
    def gather_k(x0h, x1h, x2h, idxh, o0h, o1h, o2h,
                 x0v, x1v, x2v, idxv, o0v, o1v, o2v):
        wid = lax.axis_index("s") * 2 + lax.axis_index("c")
        base = wid * _CH
        pltpu.sync_copy(x0h, x0v)
        pltpu.sync_copy(x1h, x1v)
        pltpu.sync_copy(x2h, x2v)
        pltpu.sync_copy(idxh.at[pl.ds(base, _CH)], idxv)

        def body(g, _):
            off = pl.multiple_of(g * 16, 16)
            iv = idxv[pl.ds(off, 16)]
            o0v[pl.ds(off, 16)] = plsc.load_gather(x0v, [iv])
            o1v[pl.ds(off, 16)] = plsc.load_gather(x1v, [iv])
            o2v[pl.ds(off, 16)] = plsc.load_gather(x2v, [iv])
            return _

        lax.fori_loop(0, _CH // 16, body, None)
        pltpu.sync_copy(o0v, o0h.at[pl.ds(base, _CH)])
        pltpu.sync_copy(o1v, o1h.at[pl.ds(base, _CH)])
        pltpu.sync_copy(o2v, o2h.at[pl.ds(base, _CH)])

    return gather_k(x0, x1, x2, idx_flat)


def _mlp_body(xp_ref, n0_ref, n1_ref, n2_ref, w1_ref, b1_ref, w2_ref, b2_ref,
              w3_ref, b3_ref, g_ref):
    nb = pl.program_id(1)
    xp = xp_ref[0]
    w1 = w1_ref[...]
    wd = w1[0:3, :] - w1[3:6, :]
    cb = (xp[:, 0:1] * wd[0:1, :] + xp[:, 1:2] * wd[1:2, :]
          + xp[:, 2:3] * wd[2:3, :]) + b1_ref[...]
    n0, n1, n2 = n0_ref[0], n1_ref[0], n2_ref[0]
    w2 = w2_ref[...]
    b2 = b2_ref[...]
    h1s = []
    for k in range(K):
        bn = (n0[:, k:k + 1] * w1[3:4, :] + n1[:, k:k + 1] * w1[4:5, :]
              + n2[:, k:k + 1] * w1[5:6, :])
        h1s.append(jnp.maximum(cb + bn, 0.0))
    h1 = jnp.concatenate(h1s, axis=0)
    h2 = jnp.maximum(
        jnp.dot(h1, w2, preferred_element_type=jnp.float32) + b2, 0.0)
    acc = h2[0:Q]
    for k in range(1, K):
        acc = jnp.maximum(acc, h2[k * Q:(k + 1) * Q])
    h3 = jnp.maximum(
        jnp.dot(acc, w3_ref[...], preferred_element_type=jnp.float32)
        + b3_ref[...], 0.0)
    part = jnp.max(h3, axis=0, keepdims=True)[None]

    @pl.when(nb == 0)
    def _():
        g_ref[...] = part

    @pl.when(nb != 0)
    def _():
        g_ref[...] = jnp.maximum(g_ref[...], part)


def _mlp_call(x, n0, n1, n2, w1, b1, w2, b2, w3, b3):
    full = lambda shape: pl.BlockSpec(shape, lambda b, nb: tuple(0 for _ in shape))
    return pl.pallas_call(
        _mlp_body,
        grid=(B, NB),
        in_specs=[
            pl.BlockSpec((1, Q, D), lambda b, nb: (b, nb, 0)),
            pl.BlockSpec((1, Q, K), lambda b, nb: (b, nb, 0)),
            pl.BlockSpec((1, Q, K), lambda b, nb: (b, nb, 0)),
            pl.BlockSpec((1, Q, K), lambda b, nb: (b, nb, 0)),
            full((6, 64)), full((1, 64)),
            full((64, 128)), full((1, 128)),
            full((128, 1024)), full((1, 1024)),
        ],
        out_specs=pl.BlockSpec((1, 1, 1024), lambda b, nb: (b, 0, 0)),
        out_shape=jax.ShapeDtypeStruct((B, 1, 1024), jnp.float32),
    )(x, n0, n1, n2, w1, b1, w2, b2, w3, b3)


def _tail_body(g_ref, w1_ref, b1_ref, w2_ref, b2_ref, tw_ref, tb_ref, t_ref):
    g = g_ref[...]
    g1 = jnp.maximum(
        jnp.dot(g, w1_ref[...], preferred_element_type=jnp.float32)
        + b1_ref[...], 0.0)
    g2 = jnp.maximum(
        jnp.dot(g1, w2_ref[...], preferred_element_type=jnp.float32)
        + b2_ref[...], 0.0)
    t_ref[...] = jnp.dot(g2, tw_ref[...],
                         preferred_element_type=jnp.float32) + tb_ref[...]


def _tail_call(g, w1, b1, w2, b2, tw, tb):
    full = lambda shape: pl.BlockSpec(shape, lambda: tuple(0 for _ in shape))
    return pl.pallas_call(
        _tail_body,
        grid=(),
        in_specs=[full((B, 1024)), full((1024, 512)), full((1, 512)),
                  full((512, 256)), full((1, 256)),
                  full((256, 9)), full((1, 9))],
        out_specs=full((B, 9)),
        out_shape=jax.ShapeDtypeStruct((B, 9), jnp.float32),
    )(g, w1, b1, w2, b2, tw, tb)


def _apply_body(xp_ref, t_ref, o_ref):
    xp = xp_ref[0]
    t = t_ref[0]
    o_ref[0] = jnp.dot(xp, t, preferred_element_type=jnp.float32)


def _apply_call(x, t):
    return pl.pallas_call(
        _apply_body,
        grid=(B, NB),
        in_specs=[
            pl.BlockSpec((1, Q, D), lambda b, nb: (b, nb, 0)),
            pl.BlockSpec((1, 3, 3), lambda b, nb: (b, 0, 0)),
        ],
        out_specs=pl.BlockSpec((1, Q, D), lambda b, nb: (b, nb, 0)),
        out_shape=jax.ShapeDtypeStruct((B, N, D), jnp.float32),
    )(x, t)


def kernel(inputs, conv1_w, conv1_b, conv2_w, conv2_b, conv3_w, conv3_b,
           dense1_w, dense1_b, dense2_w, dense2_b, transform_w, transform_b):
    x = inputs
    xt = jnp.transpose(x, (0, 2, 1))
    idx = _knn_call(x, xt)
    planes = jnp.transpose(x, (2, 0, 1)).reshape(3, B * N)
    n0, n1, n2 = _sc_gather(planes[0], planes[1], planes[2],
                            idx.reshape(-1))
    n0 = n0.reshape(B, N, K)
    n1 = n1.reshape(B, N, K)
    n2 = n2.reshape(B, N, K)
    g = _mlp_call(x, n0, n1, n2, conv1_w, conv1_b.reshape(1, 64),
                  conv2_w, conv2_b.reshape(1, 128),
                  conv3_w, conv3_b.reshape(1, 1024))
    t = _tail_call(g.reshape(B, 1024), dense1_w, dense1_b.reshape(1, 512),
                   dense2_w, dense2_b.reshape(1, 256),
                   transform_w, transform_b.reshape(1, 9))
    return _apply_call(x, t.reshape(B, 3, 3))


# speedup vs baseline: 1.5731x; 1.0937x over previous
import functools

import jax
import jax.numpy as jnp
from jax import lax
from jax.experimental import pallas as pl
from jax.experimental.pallas import tpu as pltpu
from jax.experimental.pallas import tpu_sc as plsc

B, N, D, K = 4, 4096, 3, 20
Q = 512
NB = N // Q
I32MAX = 0x7FFFFFFF


def _knn_body(xp_ref, xt_ref, idx_ref):
    b = pl.program_id(0)
    xp = xp_ref[0]
    xt = xt_ref[0]
    q0, q1, q2 = xp[:, 0:1], xp[:, 1:2], xp[:, 2:3]
    k0, k1, k2 = xt[0:1, :], xt[1:2, :], xt[2:3, :]
    qsq = q0 * q0 + q1 * q1 + q2 * q2
    ksq = k0 * k0 + k1 * k1 + k2 * k2
    inner = q0 * k0 + q1 * k1 + q2 * k2
    dist = (qsq + (-2.0) * inner) + ksq
    dist_c = jnp.maximum(dist, 1e-30)
    bits = lax.bitcast_convert_type(dist_c, jnp.int32)
    jcol = lax.broadcasted_iota(jnp.int32, (Q, N), 1)
    key = lax.bitcast_convert_type((bits & -4096) | jcol, jnp.float32)
    ms = []
    for _ in range(K):
        m = jnp.min(key, axis=1, keepdims=True)
        ms.append(m)
        key = jnp.where(key == m, float("inf"), key)
    sel = lax.bitcast_convert_type(jnp.concatenate(ms, axis=1), jnp.int32)
    idx_ref[0] = (sel & 4095) + b * N


def _knn_call(x, xt):
    return pl.pallas_call(
        _knn_body,
        grid=(B, NB),
        in_specs=[
            pl.BlockSpec((1, Q, D), lambda b, nb: (b, nb, 0)),
            pl.BlockSpec((1, D, N), lambda b, nb: (b, 0, 0)),
        ],
        out_specs=pl.BlockSpec((1, Q, K), lambda b, nb: (b, nb, 0)),
        out_shape=jax.ShapeDtypeStruct((B, N, K), jnp.int32),
    )(x, xt)


_NW = 32
_TOT = B * N * K
_CH = _TOT // _NW
_NF = B * N


def _sc_gather(x0, x1, x2, idx_flat):
    mesh = plsc.VectorSubcoreMesh(core_axis_name="c", subcore_axis_name="s")

    @functools.partial(
        pl.kernel,
        mesh=mesh,
        compiler_params=pltpu.CompilerParams(needs_layout_passes=False),
        out_type=[jax.ShapeDtypeStruct((_TOT,), jnp.float32)] * 3,
        scratch_types=[
            pltpu.VMEM((_NF,), jnp.float32),
            pltpu.VMEM((_NF,), jnp.float32),
            pltpu.VMEM((_NF,), jnp.float32),
            pltpu.VMEM((_CH,), jnp.int32),
            pltpu.VMEM((_CH,), jnp.float32),
            pltpu.VMEM((_CH,), jnp.float32),
            pltpu.VMEM((_CH,), jnp.float32),
        ],
    )
    def gather_k(x0h, x1h, x2h, idxh, o0h, o1h, o2h,
                 x0v, x1v, x2v, idxv, o0v, o1v, o2v):
        wid = lax.axis_index("s") * 2 + lax.axis_index("c")
        base = wid * _CH
        pltpu.sync_copy(x0h, x0v)
        pltpu.sync_copy(x1h, x1v)
        pltpu.sync_copy(x2h, x2v)
        pltpu.sync_copy(idxh.at[pl.ds(base, _CH)], idxv)

        def body(g, _):
            off = pl.multiple_of(g * 16, 16)
            iv = idxv[pl.ds(off, 16)]
            o0v[pl.ds(off, 16)] = plsc.load_gather(x0v, [iv])
            o1v[pl.ds(off, 16)] = plsc.load_gather(x1v, [iv])
            o2v[pl.ds(off, 16)] = plsc.load_gather(x2v, [iv])
            return _

        lax.fori_loop(0, _CH // 16, body, None)
        pltpu.sync_copy(o0v, o0h.at[pl.ds(base, _CH)])
        pltpu.sync_copy(o1v, o1h.at[pl.ds(base, _CH)])
        pltpu.sync_copy(o2v, o2h.at[pl.ds(base, _CH)])

    return gather_k(x0, x1, x2, idx_flat)


def _mlp_body(xp_ref, n0_ref, n1_ref, n2_ref, w1_ref, b1_ref, w2_ref, b2_ref,
              w3_ref, b3_ref, g_ref):
    nb = pl.program_id(1)
    xp = xp_ref[0]
    w1 = w1_ref[...]
    wd = w1[0:3, :] - w1[3:6, :]
    cb = jnp.dot(xp, wd, preferred_element_type=jnp.float32) + b1_ref[...]
    w2 = w2_ref[...]
    b2 = b2_ref[...]
    lane_blk = jnp.right_shift(
        lax.broadcasted_iota(jnp.int32, (K, K * 64), 1), 6)
    row = lax.broadcasted_iota(jnp.int32, (K, K * 64), 0)
    blk = lane_blk == row
    bn = None
    for d, nd_ref in ((0, n0_ref), (1, n1_ref), (2, n2_ref)):
        wt = jnp.concatenate([w1[3 + d:4 + d, :]] * K, axis=1)
        wdg = jnp.where(blk, wt, 0.0)
        p = jnp.dot(nd_ref[0], wdg, preferred_element_type=jnp.float32)
        bn = p if bn is None else bn + p
    cbt = jnp.concatenate([cb] * K, axis=1)
    h1f = jnp.maximum(bn + cbt, 0.0)
    acc = None
    for k in range(K):
        h1k = h1f[:, k * 64:(k + 1) * 64]
        h2 = jnp.maximum(
            jnp.dot(h1k, w2, preferred_element_type=jnp.float32) + b2, 0.0)
        acc = h2 if acc is None else jnp.maximum(acc, h2)
    h3 = jnp.maximum(
        jnp.dot(acc, w3_ref[...], preferred_element_type=jnp.float32)
        + b3_ref[...], 0.0)
    part = jnp.max(h3, axis=0, keepdims=True)[None]

    @pl.when(nb == 0)
    def _():
        g_ref[...] = part

    @pl.when(nb != 0)
    def _():
        g_ref[...] = jnp.maximum(g_ref[...], part)


def _mlp_call(x, n0, n1, n2, w1, b1, w2, b2, w3, b3):
    full = lambda shape: pl.BlockSpec(shape, lambda b, nb: tuple(0 for _ in shape))
    return pl.pallas_call(
        _mlp_body,
        grid=(B, NB),
        in_specs=[
            pl.BlockSpec((1, Q, D), lambda b, nb: (b, nb, 0)),
            pl.BlockSpec((1, Q, K), lambda b, nb: (b, nb, 0)),
            pl.BlockSpec((1, Q, K), lambda b, nb: (b, nb, 0)),
            pl.BlockSpec((1, Q, K), lambda b, nb: (b, nb, 0)),
            full((6, 64)), full((1, 64)),
            full((64, 128)), full((1, 128)),
            full((128, 1024)), full((1, 1024)),
        ],
        out_specs=pl.BlockSpec((1, 1, 1024), lambda b, nb: (b, 0, 0)),
        out_shape=jax.ShapeDtypeStruct((B, 1, 1024), jnp.float32),
    )(x, n0, n1, n2, w1, b1, w2, b2, w3, b3)


def _tail_body(g_ref, w1_ref, b1_ref, w2_ref, b2_ref, tw_ref, tb_ref, t_ref):
    g = g_ref[...]
    g1 = jnp.maximum(
        jnp.dot(g, w1_ref[...], preferred_element_type=jnp.float32)
        + b1_ref[...], 0.0)
    g2 = jnp.maximum(
        jnp.dot(g1, w2_ref[...], preferred_element_type=jnp.float32)
        + b2_ref[...], 0.0)
    t_ref[...] = jnp.dot(g2, tw_ref[...],
                         preferred_element_type=jnp.float32) + tb_ref[...]


def _tail_call(g, w1, b1, w2, b2, tw, tb):
    full = lambda shape: pl.BlockSpec(shape, lambda: tuple(0 for _ in shape))
    return pl.pallas_call(
        _tail_body,
        grid=(),
        in_specs=[full((B, 1024)), full((1024, 512)), full((1, 512)),
                  full((512, 256)), full((1, 256)),
                  full((256, 9)), full((1, 9))],
        out_specs=full((B, 9)),
        out_shape=jax.ShapeDtypeStruct((B, 9), jnp.float32),
    )(g, w1, b1, w2, b2, tw, tb)


def _apply_body(xp_ref, t_ref, o_ref):
    xp = xp_ref[0]
    t = t_ref[0]
    o_ref[0] = jnp.dot(xp, t, preferred_element_type=jnp.float32)


def _apply_call(x, t):
    return pl.pallas_call(
        _apply_body,
        grid=(B, NB),
        in_specs=[
            pl.BlockSpec((1, Q, D), lambda b, nb: (b, nb, 0)),
            pl.BlockSpec((1, 3, 3), lambda b, nb: (b, 0, 0)),
        ],
        out_specs=pl.BlockSpec((1, Q, D), lambda b, nb: (b, nb, 0)),
        out_shape=jax.ShapeDtypeStruct((B, N, D), jnp.float32),
    )(x, t)


def kernel(inputs, conv1_w, conv1_b, conv2_w, conv2_b, conv3_w, conv3_b,
           dense1_w, dense1_b, dense2_w, dense2_b, transform_w, transform_b):
    x = inputs
    xt = jnp.transpose(x, (0, 2, 1))
    idx = _knn_call(x, xt)
    planes = jnp.transpose(x, (2, 0, 1)).reshape(3, B * N)
    n0, n1, n2 = _sc_gather(planes[0], planes[1], planes[2],
                            idx.reshape(-1))
    n0 = n0.reshape(B, N, K)
    n1 = n1.reshape(B, N, K)
    n2 = n2.reshape(B, N, K)
    g = _mlp_call(x, n0, n1, n2, conv1_w, conv1_b.reshape(1, 64),
                  conv2_w, conv2_b.reshape(1, 128),
                  conv3_w, conv3_b.reshape(1, 1024))
    t = _tail_call(g.reshape(B, 1024), dense1_w, dense1_b.reshape(1, 512),
                   dense2_w, dense2_b.reshape(1, 256),
                   transform_w, transform_b.reshape(1, 9))
    return _apply_call(x, t.reshape(B, 3, 3))


# speedup vs baseline: 1.7344x; 1.1025x over previous
import functools

import jax
import jax.numpy as jnp
from jax import lax
from jax.experimental import pallas as pl
from jax.experimental.pallas import tpu as pltpu
from jax.experimental.pallas import tpu_sc as plsc

B, N, D, K = 4, 4096, 3, 20
Q = 512
NB = N // Q
I32MAX = 0x7FFFFFFF


def _knn_body(xp_ref, xt_ref, idx_ref):
    b = pl.program_id(0)
    xp = xp_ref[0]
    xt = xt_ref[0]
    k0, k1, k2 = xt[0:1, :], xt[1:2, :], xt[2:3, :]
    ksq = k0 * k0 + k1 * k1 + k2 * k2
    qsq = jnp.sum(xp * xp, axis=1, keepdims=True)
    xp_aug = jnp.concatenate([xp * -2.0, qsq], axis=1)
    ones = jnp.ones((1, N), jnp.float32)
    xt_aug = jnp.concatenate([xt, ones], axis=0)
    dist = jnp.dot(xp_aug, xt_aug,
                   preferred_element_type=jnp.float32) + ksq
    dist_c = jnp.maximum(dist, 1e-30)
    bits = lax.bitcast_convert_type(dist_c, jnp.int32)
    jcol = lax.broadcasted_iota(jnp.int32, (Q, N), 1)
    key = lax.bitcast_convert_type((bits & -4096) | jcol, jnp.float32)
    ms = []
    for _ in range(K):
        m = jnp.min(key, axis=1, keepdims=True)
        ms.append(m)
        key = jnp.where(key == m, float("inf"), key)
    sel = lax.bitcast_convert_type(jnp.concatenate(ms, axis=1), jnp.int32)
    idx_ref[0] = (sel & 4095) + b * N


def _knn_call(x, xt):
    return pl.pallas_call(
        _knn_body,
        grid=(B, NB),
        in_specs=[
            pl.BlockSpec((1, Q, D), lambda b, nb: (b, nb, 0)),
            pl.BlockSpec((1, D, N), lambda b, nb: (b, 0, 0)),
        ],
        out_specs=pl.BlockSpec((1, Q, K), lambda b, nb: (b, nb, 0)),
        out_shape=jax.ShapeDtypeStruct((B, N, K), jnp.int32),
    )(x, xt)


_NW = 32
_TOT = B * N * K
_CH = _TOT // _NW
_NF = B * N


def _sc_gather(x0, x1, x2, idx_flat):
    mesh = plsc.VectorSubcoreMesh(core_axis_name="c", subcore_axis_name="s")

    @functools.partial(
        pl.kernel,
        mesh=mesh,
        compiler_params=pltpu.CompilerParams(needs_layout_passes=False),
        out_type=[jax.ShapeDtypeStruct((_TOT,), jnp.float32)] * 3,
        scratch_types=[
            pltpu.VMEM((_NF,), jnp.float32),
            pltpu.VMEM((_NF,), jnp.float32),
            pltpu.VMEM((_NF,), jnp.float32),
            pltpu.VMEM((_CH,), jnp.int32),
            pltpu.VMEM((_CH,), jnp.float32),
            pltpu.VMEM((_CH,), jnp.float32),
            pltpu.VMEM((_CH,), jnp.float32),
        ],
    )
    def gather_k(x0h, x1h, x2h, idxh, o0h, o1h, o2h,
                 x0v, x1v, x2v, idxv, o0v, o1v, o2v):
        wid = lax.axis_index("s") * 2 + lax.axis_index("c")
        base = wid * _CH
        pltpu.sync_copy(x0h, x0v)
        pltpu.sync_copy(x1h, x1v)
        pltpu.sync_copy(x2h, x2v)
        pltpu.sync_copy(idxh.at[pl.ds(base, _CH)], idxv)

        def body(g, _):
            off = pl.multiple_of(g * 16, 16)
            iv = idxv[pl.ds(off, 16)]
            o0v[pl.ds(off, 16)] = plsc.load_gather(x0v, [iv])
            o1v[pl.ds(off, 16)] = plsc.load_gather(x1v, [iv])
            o2v[pl.ds(off, 16)] = plsc.load_gather(x2v, [iv])
            return _

        lax.fori_loop(0, _CH // 16, body, None)
        pltpu.sync_copy(o0v, o0h.at[pl.ds(base, _CH)])
        pltpu.sync_copy(o1v, o1h.at[pl.ds(base, _CH)])
        pltpu.sync_copy(o2v, o2h.at[pl.ds(base, _CH)])

    return gather_k(x0, x1, x2, idx_flat)


def _mlp_body(xp_ref, n0_ref, n1_ref, n2_ref, w1_ref, b1_ref, w2_ref, b2_ref,
              w3_ref, b3_ref, g_ref):
    nb = pl.program_id(1)
    xp = xp_ref[0]
    w1 = w1_ref[...]
    wd = w1[0:3, :] - w1[3:6, :]
    cb = jnp.dot(xp, wd, preferred_element_type=jnp.float32) + b1_ref[...]
    w2 = w2_ref[...]
    b2 = b2_ref[...]
    lane_blk = jnp.right_shift(
        lax.broadcasted_iota(jnp.int32, (K, K * 64), 1), 6)
    row = lax.broadcasted_iota(jnp.int32, (K, K * 64), 0)
    blk = lane_blk == row
    bn = None
    for d, nd_ref in ((0, n0_ref), (1, n1_ref), (2, n2_ref)):
        wt = jnp.concatenate([w1[3 + d:4 + d, :]] * K, axis=1)
        wdg = jnp.where(blk, wt, 0.0)
        p = jnp.dot(nd_ref[0], wdg, preferred_element_type=jnp.float32)
        bn = p if bn is None else bn + p
    cbt = jnp.concatenate([cb] * K, axis=1)
    h1f = jnp.maximum(bn + cbt, 0.0)
    acc = None
    for k in range(K):
        h1k = h1f[:, k * 64:(k + 1) * 64]
        h2 = jnp.maximum(
            jnp.dot(h1k, w2, preferred_element_type=jnp.float32) + b2, 0.0)
        acc = h2 if acc is None else jnp.maximum(acc, h2)
    h3 = jnp.maximum(
        jnp.dot(acc, w3_ref[...], preferred_element_type=jnp.float32)
        + b3_ref[...], 0.0)
    part = jnp.max(h3, axis=0, keepdims=True)[None]

    @pl.when(nb == 0)
    def _():
        g_ref[...] = part

    @pl.when(nb != 0)
    def _():
        g_ref[...] = jnp.maximum(g_ref[...], part)


def _mlp_call(x, n0, n1, n2, w1, b1, w2, b2, w3, b3):
    full = lambda shape: pl.BlockSpec(shape, lambda b, nb: tuple(0 for _ in shape))
    return pl.pallas_call(
        _mlp_body,
        grid=(B, NB),
        in_specs=[
            pl.BlockSpec((1, Q, D), lambda b, nb: (b, nb, 0)),
            pl.BlockSpec((1, Q, K), lambda b, nb: (b, nb, 0)),
            pl.BlockSpec((1, Q, K), lambda b, nb: (b, nb, 0)),
            pl.BlockSpec((1, Q, K), lambda b, nb: (b, nb, 0)),
            full((6, 64)), full((1, 64)),
            full((64, 128)), full((1, 128)),
            full((128, 1024)), full((1, 1024)),
        ],
        out_specs=pl.BlockSpec((1, 1, 1024), lambda b, nb: (b, 0, 0)),
        out_shape=jax.ShapeDtypeStruct((B, 1, 1024), jnp.float32),
    )(x, n0, n1, n2, w1, b1, w2, b2, w3, b3)


def _tail_body(g_ref, w1_ref, b1_ref, w2_ref, b2_ref, tw_ref, tb_ref, t_ref):
    g = g_ref[...]
    g1 = jnp.maximum(
        jnp.dot(g, w1_ref[...], preferred_element_type=jnp.float32)
        + b1_ref[...], 0.0)
    g2 = jnp.maximum(
        jnp.dot(g1, w2_ref[...], preferred_element_type=jnp.float32)
        + b2_ref[...], 0.0)
    t_ref[...] = jnp.dot(g2, tw_ref[...],
                         preferred_element_type=jnp.float32) + tb_ref[...]


def _tail_call(g, w1, b1, w2, b2, tw, tb):
    full = lambda shape: pl.BlockSpec(shape, lambda: tuple(0 for _ in shape))
    return pl.pallas_call(
        _tail_body,
        grid=(),
        in_specs=[full((B, 1024)), full((1024, 512)), full((1, 512)),
                  full((512, 256)), full((1, 256)),
                  full((256, 9)), full((1, 9))],
        out_specs=full((B, 9)),
        out_shape=jax.ShapeDtypeStruct((B, 9), jnp.float32),
    )(g, w1, b1, w2, b2, tw, tb)


def _apply_body(xp_ref, t_ref, o_ref):
    xp = xp_ref[0]
    t = t_ref[0]
    o_ref[0] = jnp.dot(xp, t, preferred_element_type=jnp.float32)


def _apply_call(x, t):
    return pl.pallas_call(
        _apply_body,
        grid=(B, NB),
        in_specs=[
            pl.BlockSpec((1, Q, D), lambda b, nb: (b, nb, 0)),
            pl.BlockSpec((1, 3, 3), lambda b, nb: (b, 0, 0)),
        ],
        out_specs=pl.BlockSpec((1, Q, D), lambda b, nb: (b, nb, 0)),
        out_shape=jax.ShapeDtypeStruct((B, N, D), jnp.float32),
    )(x, t)


def kernel(inputs, conv1_w, conv1_b, conv2_w, conv2_b, conv3_w, conv3_b,
           dense1_w, dense1_b, dense2_w, dense2_b, transform_w, transform_b):
    x = inputs
    xt = jnp.transpose(x, (0, 2, 1))
    idx = _knn_call(x, xt)
    planes = jnp.transpose(x, (2, 0, 1)).reshape(3, B * N)
    n0, n1, n2 = _sc_gather(planes[0], planes[1], planes[2],
                            idx.reshape(-1))
    n0 = n0.reshape(B, N, K)
    n1 = n1.reshape(B, N, K)
    n2 = n2.reshape(B, N, K)
    g = _mlp_call(x, n0, n1, n2, conv1_w, conv1_b.reshape(1, 64),
                  conv2_w, conv2_b.reshape(1, 128),
                  conv3_w, conv3_b.reshape(1, 1024))
    t = _tail_call(g.reshape(B, 1024), dense1_w, dense1_b.reshape(1, 512),
                   dense2_w, dense2_b.reshape(1, 256),
                   transform_w, transform_b.reshape(1, 9))
    return _apply_call(x, t.reshape(B, 3, 3))
